# baseline (speedup 1.0000x reference)
import functools

import jax
import jax.numpy as jnp
from jax import lax
from jax.experimental import pallas as pl
from jax.experimental.pallas import tpu as pltpu
from jax.experimental.pallas import tpu_sc as plsc

_LABELNUM = 1000
_NC = 2
_NS = 16
_NW = _NC * _NS
_BLK_ROWS = 64
_NBUF = 2
_BLK_WORDS = _BLK_ROWS * _LABELNUM


def _onehot_body(rows_per_w, idx_hbm, out_hbm, idx_v, buf, *sems):
    nblk = rows_per_w // _BLK_ROWS
    wid = lax.axis_index("s") * _NC + lax.axis_index("c")
    base_row = wid * rows_per_w

    pltpu.sync_copy(idx_hbm.at[pl.ds(base_row, rows_per_w)], idx_v)

    def zero_body(i, carry):
        buf[pl.ds(i * 16, 16)] = jnp.zeros((16,), jnp.float32)
        return carry

    lax.fori_loop(0, (_NBUF * _BLK_WORDS) // 16, zero_body, 0)

    ones = jnp.ones((16,), jnp.float32)
    zeros = jnp.zeros((16,), jnp.float32)
    rows16 = lax.iota(jnp.int32, 16)

    def positions(blk, j, p):
        labs = idx_v[pl.ds(blk * _BLK_ROWS + j * 16, 16)]
        local = (rows16 + j * 16) * _LABELNUM + labs
        return local + p * _BLK_WORDS

    copies = [None] * _NBUF
    for b in range(nblk):
        p = b % _NBUF
        if b >= _NBUF:
            copies[p].wait()
            for j in range(_BLK_ROWS // 16):
                plsc.store_scatter(buf, [positions(b - _NBUF, j, p)], zeros)
        for j in range(_BLK_ROWS // 16):
            plsc.store_scatter(buf, [positions(b, j, p)], ones)
        cp = pltpu.make_async_copy(
            buf.at[pl.ds(p * _BLK_WORDS, _BLK_WORDS)],
            out_hbm.at[pl.ds((base_row + b * _BLK_ROWS) * _LABELNUM, _BLK_WORDS)],
            sems[p],
        )
        cp.start()
        copies[p] = cp
    for p in range(_NBUF):
        copies[p].wait()


def kernel(input):
    B = input.shape[0]
    rows_per_w = B // _NW
    idx_flat = input.reshape(-1).astype(jnp.int32)
    mesh = plsc.VectorSubcoreMesh(core_axis_name="c", subcore_axis_name="s")
    sc_call = pl.kernel(
        functools.partial(_onehot_body, rows_per_w),
        out_type=jax.ShapeDtypeStruct((B * _LABELNUM,), jnp.float32),
        mesh=mesh,
        compiler_params=pltpu.CompilerParams(needs_layout_passes=False),
        scratch_types=[
            pltpu.VMEM((rows_per_w,), jnp.int32),
            pltpu.VMEM((_NBUF * _BLK_WORDS,), jnp.float32),
        ]
        + [pltpu.SemaphoreType.DMA] * _NBUF,
    )
    return sc_call(idx_flat).reshape(B, _LABELNUM)


# speedup vs baseline: 2.2074x; 2.2074x over previous
import jax
import jax.numpy as jnp
from jax import lax
from jax.experimental import pallas as pl
from jax.experimental.pallas import tpu as pltpu

_LABELNUM = 1000
_BLK = 512
_NBUF = 4


def _onehot_block(idx_ref, out_ref, scratch, sems):
    i = pl.program_id(0)
    n = pl.num_programs(0)
    p = lax.rem(i, _NBUF)

    @pl.when(i >= _NBUF)
    def _wait_prev():
        pltpu.make_async_copy(
            scratch.at[p],
            out_ref.at[pl.ds((i - _NBUF) * _BLK, _BLK)],
            sems.at[p],
        ).wait()

    idx = idx_ref[pl.ds(i * _BLK, _BLK), :]
    cols = jax.lax.broadcasted_iota(jnp.int32, (_BLK, _LABELNUM), 1)
    scratch[p] = (cols == idx).astype(jnp.float32)
    pltpu.make_async_copy(
        scratch.at[p],
        out_ref.at[pl.ds(i * _BLK, _BLK)],
        sems.at[p],
    ).start()

    @pl.when(i == n - 1)
    def _drain():
        for j in range(-_NBUF, 0):
            pltpu.make_async_copy(
                scratch.at[lax.rem(i + j + _NBUF + 1, _NBUF)],
                out_ref.at[pl.ds((i + j + 1) * _BLK, _BLK)],
                sems.at[lax.rem(i + j + _NBUF + 1, _NBUF)],
            ).wait()


def kernel(input):
    B = input.shape[0]
    idx = input.astype(jnp.int32)
    return pl.pallas_call(
        _onehot_block,
        grid=(B // _BLK,),
        in_specs=[pl.BlockSpec(memory_space=pltpu.VMEM)],
        out_specs=pl.BlockSpec(memory_space=pltpu.HBM),
        out_shape=jax.ShapeDtypeStruct((B, _LABELNUM), jnp.float32),
        scratch_shapes=[
            pltpu.VMEM((_NBUF, _BLK, _LABELNUM), jnp.float32),
            pltpu.SemaphoreType.DMA((_NBUF,)),
        ],
    )(idx)
